# Initial kernel scaffold; baseline (speedup 1.0000x reference)
import jax
import jax.numpy as jnp
from jax.experimental import pallas as pl


def kernel(k, v, rkn_score, m_k, m_v, m_u):
    raise NotImplementedError("write your pallas kernel here")


# speedup vs baseline: 7.9979x; 7.9979x over previous
import jax
import jax.numpy as jnp
from jax.experimental import pallas as pl
from jax.experimental.pallas import tpu as pltpu

M_ = 100
MP = 128
K2_ = 9
KDIM_ = 256
VDIM_ = 3
H_ = 64
W_ = 64
HW_ = H_ * W_
PAD = 72
KP_ROWS = HW_ + 2 * PAD
RBLK = 512


def _select_body(k_ref, mk_ref, src_ref, kp):
    kp[0:PAD, :] = jnp.zeros((PAD, KDIM_), jnp.float32)
    kp[PAD + HW_:, :] = jnp.zeros((PAD, KDIM_), jnp.float32)
    kp[PAD:PAD + HW_, :] = k_ref[0]

    xcol = jax.lax.broadcasted_iota(jnp.int32, (HW_, 1), 0) % W_

    acc = jnp.zeros((HW_, MP), jnp.float32)
    for t in range(K2_):
        dy, dx = t // 3 - 1, t % 3 - 1
        start = PAD + W_ * dy + dx
        sh = kp[start:start + HW_, :]
        if dx == -1:
            sh = sh * (xcol >= 1).astype(jnp.float32)
        elif dx == 1:
            sh = sh * (xcol <= W_ - 2).astype(jnp.float32)
        w = mk_ref[0, :, t, :]
        acc = acc + jax.lax.dot_general(
            sh, w, (((1,), (1,)), ((), ())),
            preferred_element_type=jnp.float32)

    mcol = jax.lax.broadcasted_iota(jnp.int32, (HW_, MP), 1)
    lm = jnp.where(mcol < M_, acc, -1e30)
    lmax = jnp.max(lm, axis=1, keepdims=True)
    denom = jnp.sum(jnp.exp(lm - lmax), axis=1, keepdims=True)
    val = 1.0 / denom
    valT = jnp.transpose(val)

    cnt_blocks = []
    for i in range(HW_ // RBLK):
        vi = val[i * RBLK:(i + 1) * RBLK, :]
        pio = (jax.lax.broadcasted_iota(jnp.int32, (RBLK, 1), 0)
               + i * RBLK)
        qio = jax.lax.broadcasted_iota(jnp.int32, (RBLK, HW_), 1)
        less = valT < vi
        tie = (valT == vi) & (qio < pio)
        cnt_blocks.append(jnp.sum(jnp.where(less | tie, 1.0, 0.0),
                                  axis=1, keepdims=True))
    rank_col = jnp.concatenate(cnt_blocks, axis=0)

    r_io = jax.lax.broadcasted_iota(jnp.int32, (HW_, MP), 1).astype(
        jnp.float32)
    q_io = jax.lax.broadcasted_iota(jnp.int32, (HW_, MP), 0).astype(
        jnp.float32)
    hit = rank_col == r_io
    srcT = jnp.sum(jnp.where(hit, q_io, 0.0), axis=0, keepdims=True)
    src_ref[0, 0, :] = srcT[0, :].astype(jnp.int32)


def _gather_body(src_smem, k_ref, v_ref, outk_ref, outv_ref, kp, vp):
    b = pl.program_id(0)
    zk = jnp.zeros((PAD, KDIM_), jnp.float32)
    zv = jnp.zeros((PAD, VDIM_), jnp.float32)
    kp[0:PAD, :] = zk
    kp[PAD + HW_:, :] = zk
    kp[PAD:PAD + HW_, :] = k_ref[0]
    vp[0:PAD, :] = zv
    vp[PAD + HW_:, :] = zv
    vp[PAD:PAD + HW_, :] = v_ref[0]

    def body(r, carry):
        s = src_smem[b * MP + r]
        xs = jax.lax.rem(s, W_)
        for t in range(K2_):
            dy, dx = t // 3 - 1, t % 3 - 1
            row = s + (PAD + W_ * dy + dx)
            if dx == -1:
                row = jnp.where(xs >= 1, row, 0)
            elif dx == 1:
                row = jnp.where(xs <= W_ - 2, row, 0)
            outk_ref[0, pl.ds(r, 1), t, :] = kp[pl.ds(row, 1), :]
            outv_ref[0, pl.ds(r, 1), t, :] = vp[pl.ds(row, 1), :]
        return carry

    jax.lax.fori_loop(0, M_, body, 0)


def kernel(k, v, rkn_score, m_k, m_v, m_u):
    del rkn_score, m_v, m_u
    B = k.shape[0]
    mk_pad = jnp.pad(m_k, ((0, 0), (0, MP - M_), (0, 0), (0, 0)))

    src = pl.pallas_call(
        _select_body,
        grid=(B,),
        in_specs=[
            pl.BlockSpec((1, HW_, KDIM_), lambda b: (b, 0, 0)),
            pl.BlockSpec((1, MP, K2_, KDIM_), lambda b: (b, 0, 0, 0)),
        ],
        out_specs=pl.BlockSpec((1, 1, MP), lambda b: (b, 0, 0)),
        out_shape=jax.ShapeDtypeStruct((B, 1, MP), jnp.int32),
        scratch_shapes=[pltpu.VMEM((KP_ROWS, KDIM_), jnp.float32)],
        compiler_params=pltpu.CompilerParams(
            dimension_semantics=("arbitrary",)),
    )(k, mk_pad)

    outk, outv = pl.pallas_call(
        _gather_body,
        grid_spec=pltpu.PrefetchScalarGridSpec(
            num_scalar_prefetch=1,
            grid=(B,),
            in_specs=[
                pl.BlockSpec((1, HW_, KDIM_), lambda b, s: (b, 0, 0)),
                pl.BlockSpec((1, HW_, VDIM_), lambda b, s: (b, 0, 0)),
            ],
            out_specs=[
                pl.BlockSpec((1, M_, K2_, KDIM_), lambda b, s: (b, 0, 0, 0)),
                pl.BlockSpec((1, M_, K2_, VDIM_), lambda b, s: (b, 0, 0, 0)),
            ],
            scratch_shapes=[
                pltpu.VMEM((KP_ROWS, KDIM_), jnp.float32),
                pltpu.VMEM((KP_ROWS, VDIM_), jnp.float32),
            ],
        ),
        out_shape=[
            jax.ShapeDtypeStruct((B, M_, K2_, KDIM_), jnp.float32),
            jax.ShapeDtypeStruct((B, M_, K2_, VDIM_), jnp.float32),
        ],
        compiler_params=pltpu.CompilerParams(
            dimension_semantics=("arbitrary",)),
    )(src.reshape(B * MP), k, v)
    return outk, outv
